# baseline (speedup 1.0000x reference)
import functools

import jax
import jax.numpy as jnp
from jax import lax
from jax.experimental import pallas as pl
from jax.experimental.pallas import tpu as pltpu
from jax.experimental.pallas import tpu_sc as plsc

B, H, L, D, Q = 8, 8, 2048, 128, 16
HL = H * L
HQ = H * Q
NW = 4 * B
NROW = 4 * B * HL
NCHUNK = D // 16


def _sc_body(k0, v0, k1, v1, l0, l1, l2, l3, pos_hbm,
             out, pos_v, rows_v, idx_v, sem):
    c = lax.axis_index("c")
    s = lax.axis_index("s")
    w = s * 2 + c
    l = w // B
    b = w % B
    base = w * HL

    for li, cref in enumerate((k0, v0, k1, v1)):
        @pl.when(l == li)
        def _(cref=cref):
            pltpu.async_copy(cref.at[pl.ds(b * HL, HL)],
                             out.at[pl.ds(base, HL)], sem)

    pltpu.sync_copy(pos_hbm.at[b], pos_v)
    for li, lref in enumerate((l0, l1, l2, l3)):
        @pl.when(l == li)
        def _(lref=lref):
            pltpu.sync_copy(lref.at[pl.ds(b * HQ, HQ)], rows_v)

    pos = pos_v[...]
    iota = lax.iota(jnp.int32, 16)

    for h in range(H):
        idx_v[pl.ds(h * Q, Q)] = pos + (base + h * L)

    m_even = (iota % 2) == 0
    m_odd = (iota % 2) == 1

    def q_step(q, g):
        pq = jnp.sum(jnp.where(iota == q, pos, 0))
        pp = jnp.sum(jnp.where(iota == q - 1, pos, 0))
        same = pq == pp
        g = jnp.where(same, g, q)

        @pl.when(same)
        def _():
            for h in range(H):
                rq = h * Q + q
                rg = h * Q + g
                for ch in range(NCHUNK):
                    msk = m_even if ch < NCHUNK // 2 else m_odd
                    fv = rows_v[rg, pl.ds(ch * 16, 16)]
                    lv = rows_v[rq, pl.ds(ch * 16, 16)]
                    bl = jnp.where(msk, lv, fv)

                    def wr(j, _, h=h, ch=ch, bl=bl):
                        rows_v[h * Q + j, pl.ds(ch * 16, 16)] = bl
                        return 0

                    lax.fori_loop(g, q + 1, wr, 0)
        return g

    lax.fori_loop(1, Q, q_step, jnp.int32(0))

    pltpu.make_async_copy(k0.at[pl.ds(b * HL, HL)],
                          out.at[pl.ds(base, HL)], sem).wait()
    pltpu.sync_copy(rows_v, out.at[idx_v])


def kernel(k_cache_0, v_cache_0, k_cache_1, v_cache_1,
           latest_k_0, latest_v_0, latest_k_1, latest_v_1,
           position_ids, seq_len):
    pos = position_ids.astype(jnp.int32)
    caches = [x.reshape(B * HL, D) for x in
              (k_cache_0, v_cache_0, k_cache_1, v_cache_1)]
    lats = [x.reshape(B * HQ, D) for x in
            (latest_k_0, latest_v_0, latest_k_1, latest_v_1)]

    mesh = plsc.VectorSubcoreMesh(core_axis_name="c", subcore_axis_name="s")
    run = functools.partial(
        pl.kernel,
        out_type=jax.ShapeDtypeStruct((NROW, D), jnp.float32),
        mesh=mesh,
        compiler_params=pltpu.CompilerParams(needs_layout_passes=False),
        scratch_types=[
            pltpu.VMEM((Q,), jnp.int32),
            pltpu.VMEM((HQ, D), jnp.float32),
            pltpu.VMEM((HQ,), jnp.int32),
            pltpu.SemaphoreType.DMA,
        ],
    )(_sc_body)

    out = run(*caches, *lats, pos)
    return out.reshape(4, B, H, L, D)


# speedup vs baseline: 38.1834x; 38.1834x over previous
import functools

import jax
import jax.numpy as jnp
from jax import lax
from jax.experimental import pallas as pl
from jax.experimental.pallas import tpu as pltpu
from jax.experimental.pallas import tpu_sc as plsc

B, H, L, D, Q = 8, 8, 2048, 128, 16
HL = H * L
HQ = H * Q
NW = 4 * B
NROW = 4 * B * HL
NCHUNK = D // 16


CH = 256
NCH = HL // CH


def _sc_body(k0, v0, k1, v1, l0, l1, l2, l3, pos_hbm,
             out, pos_v, rows_v, idx_v, buf0, buf1,
             isem0, isem1, osem0, osem1):
    c = lax.axis_index("c")
    s = lax.axis_index("s")
    w = s * 2 + c
    l = w // B
    b = w % B
    base = w * HL

    pltpu.sync_copy(pos_hbm.at[b], pos_v)
    for li, lref in enumerate((l0, l1, l2, l3)):
        @pl.when(l == li)
        def _(lref=lref):
            pltpu.sync_copy(lref.at[pl.ds(b * HQ, HQ)], rows_v)

    pos = pos_v[...]
    iota = lax.iota(jnp.int32, 16)

    for h in range(H):
        idx_v[pl.ds(h * Q, Q)] = pos + (base + h * L)

    m_even = (iota % 2) == 0
    m_odd = (iota % 2) == 1

    def q_step(q, g):
        pq = jnp.sum(jnp.where(iota == q, pos, 0))
        pp = jnp.sum(jnp.where(iota == q - 1, pos, 0))
        same = pq == pp
        g = jnp.where(same, g, q)

        @pl.when(same)
        def _():
            for h in range(H):
                rq = h * Q + q
                rg = h * Q + g
                for ch in range(NCHUNK):
                    msk = m_even if ch < NCHUNK // 2 else m_odd
                    fv = rows_v[rg, pl.ds(ch * 16, 16)]
                    lv = rows_v[rq, pl.ds(ch * 16, 16)]
                    bl = jnp.where(msk, lv, fv)

                    def wr(j, _, h=h, ch=ch, bl=bl):
                        rows_v[h * Q + j, pl.ds(ch * 16, 16)] = bl
                        return 0

                    lax.fori_loop(g, q + 1, wr, 0)
        return g

    lax.fori_loop(1, Q, q_step, jnp.int32(0))

    bufs = (buf0, buf1)
    isems = (isem0, isem1)
    osems = (osem0, osem1)
    for li, cref in enumerate((k0, v0, k1, v1)):
        @pl.when(l == li)
        def _(cref=cref):
            def t_step(t, _):
                for j in (0, 1):
                    i = 2 * t + j
                    src = cref.at[pl.ds(b * HL + i * CH, CH)]
                    dst = out.at[pl.ds(base + i * CH, CH)]

                    @pl.when(t > 0)
                    def _(j=j, dst=dst):
                        pltpu.make_async_copy(bufs[j], dst, osems[j]).wait()

                    pltpu.async_copy(src, bufs[j], isems[j])
                for j in (0, 1):
                    i = 2 * t + j
                    src = cref.at[pl.ds(b * HL + i * CH, CH)]
                    dst = out.at[pl.ds(base + i * CH, CH)]
                    pltpu.make_async_copy(src, bufs[j], isems[j]).wait()
                    pltpu.async_copy(bufs[j], dst, osems[j])
                return 0

            lax.fori_loop(0, NCH // 2, t_step, 0)
            for j in (0, 1):
                i = NCH - 2 + j
                dst = out.at[pl.ds(base + i * CH, CH)]
                pltpu.make_async_copy(bufs[j], dst, osems[j]).wait()

    pltpu.sync_copy(rows_v, out.at[idx_v])


def kernel(k_cache_0, v_cache_0, k_cache_1, v_cache_1,
           latest_k_0, latest_v_0, latest_k_1, latest_v_1,
           position_ids, seq_len):
    pos = position_ids.astype(jnp.int32)
    caches = [x.reshape(B * HL, D) for x in
              (k_cache_0, v_cache_0, k_cache_1, v_cache_1)]
    lats = [x.reshape(B * HQ, D) for x in
            (latest_k_0, latest_v_0, latest_k_1, latest_v_1)]

    mesh = plsc.VectorSubcoreMesh(core_axis_name="c", subcore_axis_name="s")
    run = functools.partial(
        pl.kernel,
        out_type=jax.ShapeDtypeStruct((NROW, D), jnp.float32),
        mesh=mesh,
        compiler_params=pltpu.CompilerParams(needs_layout_passes=False),
        scratch_types=[
            pltpu.VMEM((Q,), jnp.int32),
            pltpu.VMEM((HQ, D), jnp.float32),
            pltpu.VMEM((HQ,), jnp.int32),
            pltpu.VMEM((CH, D), jnp.float32),
            pltpu.VMEM((CH, D), jnp.float32),
            pltpu.SemaphoreType.DMA,
            pltpu.SemaphoreType.DMA,
            pltpu.SemaphoreType.DMA,
            pltpu.SemaphoreType.DMA,
        ],
    )(_sc_body)

    out = run(*caches, *lats, pos)
    return out.reshape(4, B, H, L, D)


# speedup vs baseline: 38.3013x; 1.0031x over previous
import functools

import jax
import jax.numpy as jnp
from jax import lax
from jax.experimental import pallas as pl
from jax.experimental.pallas import tpu as pltpu
from jax.experimental.pallas import tpu_sc as plsc

B, H, L, D, Q = 8, 8, 2048, 128, 16
HL = H * L
HQ = H * Q
NW = 4 * B
NROW = 4 * B * HL
NCHUNK = D // 16


CH = 128
NCH = HL // CH
NBUF = 4


def _sc_body(k0, v0, k1, v1, l0, l1, l2, l3, pos_hbm,
             out, pos_v, rows_v, idx_v, buf0, buf1, buf2, buf3,
             isem0, isem1, isem2, isem3, osem0, osem1, osem2, osem3):
    c = lax.axis_index("c")
    s = lax.axis_index("s")
    w = s * 2 + c
    l = w // B
    b = w % B
    base = w * HL

    pltpu.sync_copy(pos_hbm.at[b], pos_v)
    for li, lref in enumerate((l0, l1, l2, l3)):
        @pl.when(l == li)
        def _(lref=lref):
            pltpu.sync_copy(lref.at[pl.ds(b * HQ, HQ)], rows_v)

    pos = pos_v[...]
    iota = lax.iota(jnp.int32, 16)

    for h in range(H):
        idx_v[pl.ds(h * Q, Q)] = pos + (base + h * L)

    m_even = (iota % 2) == 0
    m_odd = (iota % 2) == 1

    def q_step(q, g):
        pq = jnp.sum(jnp.where(iota == q, pos, 0))
        pp = jnp.sum(jnp.where(iota == q - 1, pos, 0))
        same = pq == pp
        g = jnp.where(same, g, q)

        @pl.when(same)
        def _():
            for h in range(H):
                rq = h * Q + q
                rg = h * Q + g
                for ch in range(NCHUNK):
                    msk = m_even if ch < NCHUNK // 2 else m_odd
                    fv = rows_v[rg, pl.ds(ch * 16, 16)]
                    lv = rows_v[rq, pl.ds(ch * 16, 16)]
                    bl = jnp.where(msk, lv, fv)

                    def wr(j, _, h=h, ch=ch, bl=bl):
                        rows_v[h * Q + j, pl.ds(ch * 16, 16)] = bl
                        return 0

                    lax.fori_loop(g, q + 1, wr, 0)
        return g

    lax.fori_loop(1, Q, q_step, jnp.int32(0))

    bufs = (buf0, buf1, buf2, buf3)
    isems = (isem0, isem1, isem2, isem3)
    osems = (osem0, osem1, osem2, osem3)
    for li, cref in enumerate((k0, v0, k1, v1)):
        @pl.when(l == li)
        def _(cref=cref):
            def t_step(t, _):
                for j in range(NBUF):
                    i = NBUF * t + j
                    src = cref.at[pl.ds(b * HL + i * CH, CH)]
                    dst = out.at[pl.ds(base + i * CH, CH)]

                    @pl.when(t > 0)
                    def _(j=j, dst=dst):
                        pltpu.make_async_copy(bufs[j], dst, osems[j]).wait()

                    pltpu.async_copy(src, bufs[j], isems[j])
                for j in range(NBUF):
                    i = NBUF * t + j
                    src = cref.at[pl.ds(b * HL + i * CH, CH)]
                    dst = out.at[pl.ds(base + i * CH, CH)]
                    pltpu.make_async_copy(src, bufs[j], isems[j]).wait()
                    pltpu.async_copy(bufs[j], dst, osems[j])
                return 0

            lax.fori_loop(0, NCH // NBUF, t_step, 0)
            for j in range(NBUF):
                i = NCH - NBUF + j
                dst = out.at[pl.ds(base + i * CH, CH)]
                pltpu.make_async_copy(bufs[j], dst, osems[j]).wait()

    pltpu.sync_copy(rows_v, out.at[idx_v])


def kernel(k_cache_0, v_cache_0, k_cache_1, v_cache_1,
           latest_k_0, latest_v_0, latest_k_1, latest_v_1,
           position_ids, seq_len):
    pos = position_ids.astype(jnp.int32)
    caches = [x.reshape(B * HL, D) for x in
              (k_cache_0, v_cache_0, k_cache_1, v_cache_1)]
    lats = [x.reshape(B * HQ, D) for x in
            (latest_k_0, latest_v_0, latest_k_1, latest_v_1)]

    mesh = plsc.VectorSubcoreMesh(core_axis_name="c", subcore_axis_name="s")
    run = functools.partial(
        pl.kernel,
        out_type=jax.ShapeDtypeStruct((NROW, D), jnp.float32),
        mesh=mesh,
        compiler_params=pltpu.CompilerParams(needs_layout_passes=False),
        scratch_types=[
            pltpu.VMEM((Q,), jnp.int32),
            pltpu.VMEM((HQ, D), jnp.float32),
            pltpu.VMEM((HQ,), jnp.int32),
            pltpu.VMEM((CH, D), jnp.float32),
            pltpu.VMEM((CH, D), jnp.float32),
            pltpu.VMEM((CH, D), jnp.float32),
            pltpu.VMEM((CH, D), jnp.float32),
            pltpu.SemaphoreType.DMA,
            pltpu.SemaphoreType.DMA,
            pltpu.SemaphoreType.DMA,
            pltpu.SemaphoreType.DMA,
            pltpu.SemaphoreType.DMA,
            pltpu.SemaphoreType.DMA,
            pltpu.SemaphoreType.DMA,
            pltpu.SemaphoreType.DMA,
        ],
    )(_sc_body)

    out = run(*caches, *lats, pos)
    return out.reshape(4, B, H, L, D)


# speedup vs baseline: 38.7465x; 1.0116x over previous
import functools

import jax
import jax.numpy as jnp
from jax import lax
from jax.experimental import pallas as pl
from jax.experimental.pallas import tpu as pltpu
from jax.experimental.pallas import tpu_sc as plsc

B, H, L, D, Q = 8, 8, 2048, 128, 16
HL = H * L
HQ = H * Q
NW = 4 * B
NROW = 4 * B * HL
NCHUNK = D // 16


CH = 128
NCH = HL // CH
NBUF = 4


def _sc_body(k0, v0, k1, v1, l0, l1, l2, l3, pos_hbm,
             out, pos_v, rows_v, idx_v, buf0, buf1, buf2, buf3,
             psem, rsem,
             isem0, isem1, isem2, isem3, osem0, osem1, osem2, osem3):
    c = lax.axis_index("c")
    s = lax.axis_index("s")
    w = s * 2 + c
    l = w // B
    b = w % B
    base = w * HL
    bufs = (buf0, buf1, buf2, buf3)
    isems = (isem0, isem1, isem2, isem3)
    osems = (osem0, osem1, osem2, osem3)
    T = NCH // NBUF

    for li, (cref, lref) in enumerate(((k0, l0), (v0, l1), (k1, l2), (v1, l3))):
        @pl.when(l == li)
        def _(cref=cref, lref=lref):
            for j in range(NBUF):
                pltpu.async_copy(cref.at[pl.ds(b * HL + j * CH, CH)],
                                 bufs[j], isems[j])
            pltpu.async_copy(lref.at[pl.ds(b * HQ, HQ)], rows_v, rsem)
    pltpu.async_copy(pos_hbm.at[b], pos_v, psem)

    pltpu.make_async_copy(pos_hbm.at[b], pos_v, psem).wait()
    pos = pos_v[...]
    iota = lax.iota(jnp.int32, 16)
    for h in range(H):
        idx_v[pl.ds(h * Q, Q)] = pos + (base + h * L)

    prev = plsc.load_gather(pos_v, [jnp.maximum(iota - 1, 0)])
    dup_v = jnp.logical_and(pos == prev, iota > 0)
    ndup = plsc.all_reduce_population_count(dup_v)
    if ndup.shape:
        ndup = jnp.sum(jnp.where(iota == 0, ndup, 0))
    pltpu.make_async_copy(l0.at[pl.ds(b * HQ, HQ)], rows_v, rsem).wait()

    m_even = (iota % 2) == 0
    m_odd = (iota % 2) == 1

    @pl.when(ndup > 0)
    def _():
        def q_step(q, g):
            pq = jnp.sum(jnp.where(iota == q, pos, 0))
            pp = jnp.sum(jnp.where(iota == q - 1, pos, 0))
            same = pq == pp
            g = jnp.where(same, g, q)

            @pl.when(same)
            def _():
                for h in range(H):
                    rq = h * Q + q
                    rg = h * Q + g
                    for ch in range(NCHUNK):
                        msk = m_even if ch < NCHUNK // 2 else m_odd
                        fv = rows_v[rg, pl.ds(ch * 16, 16)]
                        lv = rows_v[rq, pl.ds(ch * 16, 16)]
                        bl = jnp.where(msk, lv, fv)

                        def wr(j, _, h=h, ch=ch, bl=bl):
                            rows_v[h * Q + j, pl.ds(ch * 16, 16)] = bl
                            return 0

                        lax.fori_loop(g, q + 1, wr, 0)
            return g

        lax.fori_loop(1, Q, q_step, jnp.int32(0))

    for li, cref in enumerate((k0, v0, k1, v1)):
        @pl.when(l == li)
        def _(cref=cref):
            def t_step(t, _):
                for j in range(NBUF):
                    i = NBUF * t + j
                    src = cref.at[pl.ds(b * HL + i * CH, CH)]
                    dst = out.at[pl.ds(base + i * CH, CH)]
                    pltpu.make_async_copy(src, bufs[j], isems[j]).wait()
                    pltpu.async_copy(bufs[j], dst, osems[j])
                for j in range(NBUF):
                    @pl.when(t + 1 < T)
                    def _(j=j, t=t):
                        i_old = NBUF * t + j
                        i_new = NBUF * (t + 1) + j
                        dst_old = out.at[pl.ds(base + i_old * CH, CH)]
                        pltpu.make_async_copy(bufs[j], dst_old, osems[j]).wait()
                        pltpu.async_copy(cref.at[pl.ds(b * HL + i_new * CH, CH)],
                                         bufs[j], isems[j])
                return 0

            lax.fori_loop(0, T, t_step, 0)
            for j in range(NBUF):
                i = NCH - NBUF + j
                dst = out.at[pl.ds(base + i * CH, CH)]
                pltpu.make_async_copy(bufs[j], dst, osems[j]).wait()

    pltpu.sync_copy(rows_v, out.at[idx_v])


def kernel(k_cache_0, v_cache_0, k_cache_1, v_cache_1,
           latest_k_0, latest_v_0, latest_k_1, latest_v_1,
           position_ids, seq_len):
    pos = position_ids.astype(jnp.int32)
    caches = [x.reshape(B * HL, D) for x in
              (k_cache_0, v_cache_0, k_cache_1, v_cache_1)]
    lats = [x.reshape(B * HQ, D) for x in
            (latest_k_0, latest_v_0, latest_k_1, latest_v_1)]

    mesh = plsc.VectorSubcoreMesh(core_axis_name="c", subcore_axis_name="s")
    run = functools.partial(
        pl.kernel,
        out_type=jax.ShapeDtypeStruct((NROW, D), jnp.float32),
        mesh=mesh,
        compiler_params=pltpu.CompilerParams(needs_layout_passes=False),
        scratch_types=[
            pltpu.VMEM((Q,), jnp.int32),
            pltpu.VMEM((HQ, D), jnp.float32),
            pltpu.VMEM((HQ,), jnp.int32),
            pltpu.VMEM((CH, D), jnp.float32),
            pltpu.VMEM((CH, D), jnp.float32),
            pltpu.VMEM((CH, D), jnp.float32),
            pltpu.VMEM((CH, D), jnp.float32),
            pltpu.SemaphoreType.DMA,
            pltpu.SemaphoreType.DMA,
            pltpu.SemaphoreType.DMA,
            pltpu.SemaphoreType.DMA,
            pltpu.SemaphoreType.DMA,
            pltpu.SemaphoreType.DMA,
            pltpu.SemaphoreType.DMA,
            pltpu.SemaphoreType.DMA,
            pltpu.SemaphoreType.DMA,
            pltpu.SemaphoreType.DMA,
        ],
    )(_sc_body)

    out = run(*caches, *lats, pos)
    return out.reshape(4, B, H, L, D)


# speedup vs baseline: 39.1843x; 1.0113x over previous
import functools

import jax
import jax.numpy as jnp
from jax import lax
from jax.experimental import pallas as pl
from jax.experimental.pallas import tpu as pltpu
from jax.experimental.pallas import tpu_sc as plsc

B, H, L, D, Q = 8, 8, 2048, 128, 16
HL = H * L
HQ = H * Q
NW = 4 * B
NROW = 4 * B * HL
NCHUNK = D // 16


CH = 128
NBUF = 4
CH2 = 128
NSP = 24
SPROW = NSP * CH2
NCH = (HL - SPROW) // CH
CAD = 2


def _sc_body(k0, v0, k1, v1, l0, l1, l2, l3, pos_hbm,
             out, pos_v, rows_v, idx_v, buf0, buf1, buf2, buf3, sp,
             psem, rsem,
             isem0, isem1, isem2, isem3, osem0, osem1, osem2, osem3,
             spi0, spi1, spo0, spo1):
    c = lax.axis_index("c")
    s = lax.axis_index("s")
    w = s * 2 + c
    l = w // B
    b = w % B
    base = w * HL
    bufs = (buf0, buf1, buf2, buf3)
    isems = (isem0, isem1, isem2, isem3)
    osems = (osem0, osem1, osem2, osem3)
    T = NCH // NBUF

    for li, (cref, lref) in enumerate(((k0, l0), (v0, l1), (k1, l2), (v1, l3))):
        @pl.when(l == li)
        def _(cref=cref, lref=lref):
            for j in range(NBUF):
                pltpu.async_copy(cref.at[pl.ds(b * HL + SPROW + j * CH, CH)],
                                 bufs[j], isems[j])
            pltpu.async_copy(lref.at[pl.ds(b * HQ, HQ)], rows_v, rsem)
    pltpu.async_copy(pos_hbm.at[b], pos_v, psem)

    pltpu.make_async_copy(pos_hbm.at[b], pos_v, psem).wait()
    pos = pos_v[...]
    iota = lax.iota(jnp.int32, 16)
    for h in range(H):
        idx_v[pl.ds(h * Q, Q)] = pos + (base + h * L)

    prev = plsc.load_gather(pos_v, [jnp.maximum(iota - 1, 0)])
    dup_v = jnp.logical_and(pos == prev, iota > 0)
    ndup = plsc.all_reduce_population_count(dup_v)
    if ndup.shape:
        ndup = jnp.sum(jnp.where(iota == 0, ndup, 0))
    pltpu.make_async_copy(l0.at[pl.ds(b * HQ, HQ)], rows_v, rsem).wait()

    m_even = (iota % 2) == 0
    m_odd = (iota % 2) == 1

    @pl.when(ndup > 0)
    def _():
        def q_step(q, g):
            pq = jnp.sum(jnp.where(iota == q, pos, 0))
            pp = jnp.sum(jnp.where(iota == q - 1, pos, 0))
            same = pq == pp
            g = jnp.where(same, g, q)

            @pl.when(same)
            def _():
                for h in range(H):
                    rq = h * Q + q
                    rg = h * Q + g
                    for ch in range(NCHUNK):
                        msk = m_even if ch < NCHUNK // 2 else m_odd
                        fv = rows_v[rg, pl.ds(ch * 16, 16)]
                        lv = rows_v[rq, pl.ds(ch * 16, 16)]
                        bl = jnp.where(msk, lv, fv)

                        def wr(j, _, h=h, ch=ch, bl=bl):
                            rows_v[h * Q + j, pl.ds(ch * 16, 16)] = bl
                            return 0

                        lax.fori_loop(g, q + 1, wr, 0)
            return g

        lax.fori_loop(1, Q, q_step, jnp.int32(0))

    spbufs = (sp.at[s, 0], sp.at[s, 1])
    spisems = (spi0, spi1)
    sposems = (spo0, spo1)
    for li, cref in enumerate((k0, v0, k1, v1)):
        @pl.when(l == li)
        def _(cref=cref):
            def t_step(t, _):
                u = t // CAD
                for j in range(2):
                    c = 2 * u + j
                    src = cref.at[pl.ds(b * HL + c * CH2, CH2)]
                    dst = out.at[pl.ds(base + c * CH2, CH2)]

                    @pl.when(jnp.logical_and(t % CAD == 0, u < NSP // 2))
                    def _(j=j, u=u, src=src):
                        @pl.when(u > 0)
                        def _():
                            c_old = 2 * (u - 1) + j
                            dst_old = out.at[pl.ds(base + c_old * CH2, CH2)]
                            pltpu.make_async_copy(spbufs[j], dst_old,
                                                  sposems[j]).wait()
                        pltpu.async_copy(src, spbufs[j], spisems[j])

                    @pl.when(jnp.logical_and(t % CAD == 1, u < NSP // 2))
                    def _(j=j, src=src, dst=dst):
                        pltpu.make_async_copy(src, spbufs[j], spisems[j]).wait()
                        pltpu.async_copy(spbufs[j], dst, sposems[j])
                for j in range(NBUF):
                    i = NBUF * t + j
                    src = cref.at[pl.ds(b * HL + SPROW + i * CH, CH)]
                    dst = out.at[pl.ds(base + SPROW + i * CH, CH)]
                    pltpu.make_async_copy(src, bufs[j], isems[j]).wait()
                    pltpu.async_copy(bufs[j], dst, osems[j])
                for j in range(NBUF):
                    @pl.when(t + 1 < T)
                    def _(j=j, t=t):
                        i_old = NBUF * t + j
                        i_new = NBUF * (t + 1) + j
                        dst_old = out.at[pl.ds(base + SPROW + i_old * CH, CH)]
                        pltpu.make_async_copy(bufs[j], dst_old, osems[j]).wait()
                        pltpu.async_copy(
                            cref.at[pl.ds(b * HL + SPROW + i_new * CH, CH)],
                            bufs[j], isems[j])
                return 0

            lax.fori_loop(0, T, t_step, 0)
            for j in range(NBUF):
                i = NCH - NBUF + j
                dst = out.at[pl.ds(base + SPROW + i * CH, CH)]
                pltpu.make_async_copy(bufs[j], dst, osems[j]).wait()
            for j in range(2):
                c = NSP - 2 + j
                dst = out.at[pl.ds(base + c * CH2, CH2)]
                pltpu.make_async_copy(spbufs[j], dst, sposems[j]).wait()

    pltpu.sync_copy(rows_v, out.at[idx_v])


def kernel(k_cache_0, v_cache_0, k_cache_1, v_cache_1,
           latest_k_0, latest_v_0, latest_k_1, latest_v_1,
           position_ids, seq_len):
    pos = position_ids.astype(jnp.int32)
    caches = [x.reshape(B * HL, D) for x in
              (k_cache_0, v_cache_0, k_cache_1, v_cache_1)]
    lats = [x.reshape(B * HQ, D) for x in
            (latest_k_0, latest_v_0, latest_k_1, latest_v_1)]

    mesh = plsc.VectorSubcoreMesh(core_axis_name="c", subcore_axis_name="s")
    run = functools.partial(
        pl.kernel,
        out_type=jax.ShapeDtypeStruct((NROW, D), jnp.float32),
        mesh=mesh,
        compiler_params=pltpu.CompilerParams(needs_layout_passes=False),
        scratch_types=[
            pltpu.VMEM((Q,), jnp.int32),
            pltpu.VMEM((HQ, D), jnp.float32),
            pltpu.VMEM((HQ,), jnp.int32),
            pltpu.VMEM((CH, D), jnp.float32),
            pltpu.VMEM((CH, D), jnp.float32),
            pltpu.VMEM((CH, D), jnp.float32),
            pltpu.VMEM((CH, D), jnp.float32),
            pltpu.VMEM_SHARED((16, 2, CH2, D), jnp.float32),
            pltpu.SemaphoreType.DMA,
            pltpu.SemaphoreType.DMA,
            pltpu.SemaphoreType.DMA,
            pltpu.SemaphoreType.DMA,
            pltpu.SemaphoreType.DMA,
            pltpu.SemaphoreType.DMA,
            pltpu.SemaphoreType.DMA,
            pltpu.SemaphoreType.DMA,
            pltpu.SemaphoreType.DMA,
            pltpu.SemaphoreType.DMA,
            pltpu.SemaphoreType.DMA,
            pltpu.SemaphoreType.DMA,
            pltpu.SemaphoreType.DMA,
            pltpu.SemaphoreType.DMA,
        ],
    )(_sc_body)

    out = run(*caches, *lats, pos)
    return out.reshape(4, B, H, L, D)


# speedup vs baseline: 42.0472x; 1.0731x over previous
import functools

import jax
import jax.numpy as jnp
from jax import lax
from jax.experimental import pallas as pl
from jax.experimental.pallas import tpu as pltpu
from jax.experimental.pallas import tpu_sc as plsc

B, H, L, D, Q = 8, 8, 2048, 128, 16
HQ = H * Q
NCHUNK = D // 16


def _rows_body(l0, l1, l2, l3, pos_hbm, rout, pos_v, rows_v, psem, rsem):
    c = lax.axis_index("c")
    s = lax.axis_index("s")
    w = s * 2 + c
    l = w // B
    b = w % B

    for li, lref in enumerate((l0, l1, l2, l3)):
        @pl.when(l == li)
        def _(lref=lref):
            pltpu.async_copy(lref.at[pl.ds(b * HQ, HQ)], rows_v, rsem)
    pltpu.async_copy(pos_hbm.at[b], pos_v, psem)
    pltpu.make_async_copy(pos_hbm.at[b], pos_v, psem).wait()

    pos = pos_v[...]
    iota = lax.iota(jnp.int32, 16)
    prev = plsc.load_gather(pos_v, [jnp.maximum(iota - 1, 0)])
    dup_v = jnp.logical_and(pos == prev, iota > 0)
    ndup = plsc.all_reduce_population_count(dup_v)
    if ndup.shape:
        ndup = jnp.sum(jnp.where(iota == 0, ndup, 0))
    pltpu.make_async_copy(l0.at[pl.ds(b * HQ, HQ)], rows_v, rsem).wait()

    m_even = (iota % 2) == 0
    m_odd = (iota % 2) == 1

    @pl.when(ndup > 0)
    def _():
        def q_step(q, g):
            pq = jnp.sum(jnp.where(iota == q, pos, 0))
            pp = jnp.sum(jnp.where(iota == q - 1, pos, 0))
            same = pq == pp
            g = jnp.where(same, g, q)

            @pl.when(same)
            def _():
                for h in range(H):
                    rq = h * Q + q
                    rg = h * Q + g
                    for ch in range(NCHUNK):
                        msk = m_even if ch < NCHUNK // 2 else m_odd
                        fv = rows_v[rg, pl.ds(ch * 16, 16)]
                        lv = rows_v[rq, pl.ds(ch * 16, 16)]
                        bl = jnp.where(msk, lv, fv)

                        def wr(j, _, h=h, ch=ch, bl=bl):
                            rows_v[h * Q + j, pl.ds(ch * 16, 16)] = bl
                            return 0

                        lax.fori_loop(g, q + 1, wr, 0)
            return g

        lax.fori_loop(1, Q, q_step, jnp.int32(0))

    pltpu.sync_copy(rows_v, rout.at[pl.ds(w * HQ, HQ)])


def _apply_body(pos_ref, k0, v0, k1, v1, rows, out_ref):
    b = pl.program_id(0)
    for li, cref in enumerate((k0, v0, k1, v1)):
        out_ref[li, 0, 0] = cref[0, 0]

        def q_body(q, _, li=li):
            row = pos_ref[b, q]
            out_ref[li, 0, 0, pl.ds(row, 1), :] = rows[li, 0, 0, pl.ds(q, 1), :]
            return 0

        lax.fori_loop(0, Q, q_body, 0)


def kernel(k_cache_0, v_cache_0, k_cache_1, v_cache_1,
           latest_k_0, latest_v_0, latest_k_1, latest_v_1,
           position_ids, seq_len):
    pos = position_ids.astype(jnp.int32)
    lats = [x.reshape(B * HQ, D) for x in
            (latest_k_0, latest_v_0, latest_k_1, latest_v_1)]

    mesh = plsc.VectorSubcoreMesh(core_axis_name="c", subcore_axis_name="s")
    rows_flat = functools.partial(
        pl.kernel,
        out_type=jax.ShapeDtypeStruct((4 * B * HQ, D), jnp.float32),
        mesh=mesh,
        compiler_params=pltpu.CompilerParams(needs_layout_passes=False),
        scratch_types=[
            pltpu.VMEM((Q,), jnp.int32),
            pltpu.VMEM((HQ, D), jnp.float32),
            pltpu.SemaphoreType.DMA,
            pltpu.SemaphoreType.DMA,
        ],
    )(_rows_body)(*lats, pos)
    rows = rows_flat.reshape(4, B, H, Q, D)

    cache_spec = pl.BlockSpec((1, 1, L, D), lambda b, h, *_: (b, h, 0, 0))
    rows_spec = pl.BlockSpec((4, 1, 1, Q, D), lambda b, h, *_: (0, b, h, 0, 0))
    out_spec = pl.BlockSpec((4, 1, 1, L, D), lambda b, h, *_: (0, b, h, 0, 0))

    grid_spec = pltpu.PrefetchScalarGridSpec(
        num_scalar_prefetch=1,
        grid=(B, H),
        in_specs=[cache_spec] * 4 + [rows_spec],
        out_specs=out_spec,
    )

    return pl.pallas_call(
        _apply_body,
        grid_spec=grid_spec,
        out_shape=jax.ShapeDtypeStruct((4, B, H, L, D), jnp.float32),
        compiler_params=pltpu.CompilerParams(
            dimension_semantics=("arbitrary", "arbitrary"),
        ),
    )(pos, k_cache_0, v_cache_0, k_cache_1, v_cache_1, rows)


# speedup vs baseline: 42.5423x; 1.0118x over previous
import functools

import jax
import jax.numpy as jnp
from jax import lax
from jax.experimental import pallas as pl
from jax.experimental.pallas import tpu as pltpu
from jax.experimental.pallas import tpu_sc as plsc

B, H, L, D, Q = 8, 8, 2048, 128, 16
HQ = H * Q
NCHUNK = D // 16


def _rows_body(l0, l1, l2, l3, pos_hbm, rout, pos_v, rows_v, psem, rsem):
    c = lax.axis_index("c")
    s = lax.axis_index("s")
    w = s * 2 + c
    l = w // B
    b = w % B

    for li, lref in enumerate((l0, l1, l2, l3)):
        @pl.when(l == li)
        def _(lref=lref):
            pltpu.async_copy(lref.at[pl.ds(b * HQ, HQ)], rows_v, rsem)
    pltpu.async_copy(pos_hbm.at[b], pos_v, psem)
    pltpu.make_async_copy(pos_hbm.at[b], pos_v, psem).wait()

    pos = pos_v[...]
    iota = lax.iota(jnp.int32, 16)
    prev = plsc.load_gather(pos_v, [jnp.maximum(iota - 1, 0)])
    dup_v = jnp.logical_and(pos == prev, iota > 0)
    ndup = plsc.all_reduce_population_count(dup_v)
    if ndup.shape:
        ndup = jnp.sum(jnp.where(iota == 0, ndup, 0))
    pltpu.make_async_copy(l0.at[pl.ds(b * HQ, HQ)], rows_v, rsem).wait()

    m_even = (iota % 2) == 0
    m_odd = (iota % 2) == 1

    @pl.when(ndup > 0)
    def _():
        def q_step(q, g):
            pq = jnp.sum(jnp.where(iota == q, pos, 0))
            pp = jnp.sum(jnp.where(iota == q - 1, pos, 0))
            same = pq == pp
            g = jnp.where(same, g, q)

            @pl.when(same)
            def _():
                for h in range(H):
                    rq = h * Q + q
                    rg = h * Q + g
                    for ch in range(NCHUNK):
                        msk = m_even if ch < NCHUNK // 2 else m_odd
                        fv = rows_v[rg, pl.ds(ch * 16, 16)]
                        lv = rows_v[rq, pl.ds(ch * 16, 16)]
                        bl = jnp.where(msk, lv, fv)

                        def wr(j, _, h=h, ch=ch, bl=bl):
                            rows_v[h * Q + j, pl.ds(ch * 16, 16)] = bl
                            return 0

                        lax.fori_loop(g, q + 1, wr, 0)
            return g

        lax.fori_loop(1, Q, q_step, jnp.int32(0))

    pltpu.sync_copy(rows_v, rout.at[pl.ds(w * HQ, HQ)])


HB = 2


def _apply_body(pos_ref, k0, v0, k1, v1, rows, out_ref):
    b = pl.program_id(0)
    for li, cref in enumerate((k0, v0, k1, v1)):
        for hh in range(HB):
            out_ref[li, 0, hh] = cref[0, hh]

            def q_body(q, _, li=li, hh=hh):
                row = pos_ref[b, q]
                out_ref[li, 0, hh, pl.ds(row, 1), :] = (
                    rows[li, 0, hh, pl.ds(q, 1), :])
                return 0

            lax.fori_loop(0, Q, q_body, 0)


def kernel(k_cache_0, v_cache_0, k_cache_1, v_cache_1,
           latest_k_0, latest_v_0, latest_k_1, latest_v_1,
           position_ids, seq_len):
    pos = position_ids.astype(jnp.int32)
    lats = [x.reshape(B * HQ, D) for x in
            (latest_k_0, latest_v_0, latest_k_1, latest_v_1)]

    mesh = plsc.VectorSubcoreMesh(core_axis_name="c", subcore_axis_name="s")
    rows_flat = functools.partial(
        pl.kernel,
        out_type=jax.ShapeDtypeStruct((4 * B * HQ, D), jnp.float32),
        mesh=mesh,
        compiler_params=pltpu.CompilerParams(needs_layout_passes=False),
        scratch_types=[
            pltpu.VMEM((Q,), jnp.int32),
            pltpu.VMEM((HQ, D), jnp.float32),
            pltpu.SemaphoreType.DMA,
            pltpu.SemaphoreType.DMA,
        ],
    )(_rows_body)(*lats, pos)
    rows = rows_flat.reshape(4, B, H, Q, D)

    cache_spec = pl.BlockSpec((1, HB, L, D), lambda b, h, *_: (b, h, 0, 0))
    rows_spec = pl.BlockSpec((4, 1, HB, Q, D),
                             lambda b, h, *_: (0, b, h, 0, 0))
    out_spec = pl.BlockSpec((4, 1, HB, L, D),
                            lambda b, h, *_: (0, b, h, 0, 0))

    grid_spec = pltpu.PrefetchScalarGridSpec(
        num_scalar_prefetch=1,
        grid=(B, H // HB),
        in_specs=[cache_spec] * 4 + [rows_spec],
        out_specs=out_spec,
    )

    return pl.pallas_call(
        _apply_body,
        grid_spec=grid_spec,
        out_shape=jax.ShapeDtypeStruct((4, B, H, L, D), jnp.float32),
        compiler_params=pltpu.CompilerParams(
            dimension_semantics=("arbitrary", "arbitrary"),
        ),
    )(pos, k_cache_0, v_cache_0, k_cache_1, v_cache_1, rows)
